# Initial kernel scaffold; baseline (speedup 1.0000x reference)
import jax
import jax.numpy as jnp
from jax.experimental import pallas as pl


def kernel(x, edge_index, edge_attr, weight, root, bias):
    raise NotImplementedError("write your pallas kernel here")


# speedup vs baseline: 4.5894x; 4.5894x over previous
import functools

import jax
import jax.numpy as jnp
from jax import lax
from jax.experimental import pallas as pl
from jax.experimental.pallas import tpu as pltpu
from jax.experimental.pallas import tpu_sc as plsc

N_NODES = 10000
D = 128
N_PAD = 10240
ROWS_PER_TILE = N_PAD // 16
DEG_W = 16
B = 128
NUM_CORES = 2
NUM_SUBCORES = 16
TILES = NUM_CORES * NUM_SUBCORES


CH = 8


def _sc_phase(nb):
    mesh = plsc.VectorSubcoreMesh(core_axis_name="c", subcore_axis_name="s")

    @functools.partial(
        pl.kernel,
        out_type=[
            jax.ShapeDtypeStruct((NUM_CORES, N_PAD, D), jnp.float32),
            jax.ShapeDtypeStruct((NUM_CORES, N_PAD, DEG_W), jnp.float32),
        ],
        mesh=mesh,
        scratch_types=[
            pltpu.VMEM((CH, B), jnp.int32),
            pltpu.VMEM((CH, B), jnp.int32),
            pltpu.VMEM((B, D), jnp.float32),
            pltpu.VMEM((B, DEG_W), jnp.float32),
            pltpu.VMEM((ROWS_PER_TILE, DEG_W), jnp.float32),
            pltpu.VMEM_SHARED((N_PAD, D), jnp.float32),
            pltpu.VMEM_SHARED((N_PAD, DEG_W), jnp.float32),
            pltpu.SemaphoreType.DMA,
        ],
        compiler_params=pltpu.CompilerParams(use_tc_tiling_on_sc=False),
    )
    def sc_kernel(x_hbm, src_hbm, dst_hbm, acc_hbm, deg_hbm,
                  src_c, dst_c, rows_v, ones_v, degbuf, acc_sh, deg_sh, sem):
        cid = lax.axis_index("c")
        sid = lax.axis_index("s")
        wid = cid * NUM_SUBCORES + sid
        rowbase = sid * ROWS_PER_TILE

        zeros16 = jnp.zeros((16,), jnp.float32)
        ones16 = jnp.ones((16,), jnp.float32)

        @pl.loop(0, B)
        def _(i):
            ones_v[i, :] = ones16
            for k in range(D // 16):
                rows_v[i, pl.ds(k * 16, 16)] = zeros16

        @pl.loop(0, ROWS_PER_TILE)
        def _(i):
            degbuf[i, :] = zeros16

        for c in range(ROWS_PER_TILE // B):
            pltpu.sync_copy(rows_v, acc_sh.at[pl.ds(rowbase + c * B, B)])
        pltpu.sync_copy(degbuf, deg_sh.at[pl.ds(rowbase, ROWS_PER_TILE)])

        plsc.subcore_barrier()

        @pl.loop(0, nb // CH)
        def _(o):
            pltpu.sync_copy(src_hbm.at[wid, pl.ds(o * CH, CH)], src_c)
            pltpu.sync_copy(dst_hbm.at[wid, pl.ds(o * CH, CH)], dst_c)
            for k in range(CH):
                pltpu.async_copy(x_hbm.at[src_c.at[k]], rows_v, sem).wait()
                pltpu.sync_copy(rows_v, acc_sh.at[dst_c.at[k]], add=True)
                pltpu.sync_copy(ones_v, deg_sh.at[dst_c.at[k]], add=True)

        plsc.subcore_barrier()

        for c in range(ROWS_PER_TILE // B):
            pltpu.sync_copy(acc_sh.at[pl.ds(rowbase + c * B, B)], rows_v)
            pltpu.sync_copy(rows_v, acc_hbm.at[cid, pl.ds(rowbase + c * B, B)])
        pltpu.sync_copy(deg_sh.at[pl.ds(rowbase, ROWS_PER_TILE)], degbuf)
        pltpu.sync_copy(degbuf, deg_hbm.at[cid, pl.ds(rowbase, ROWS_PER_TILE)])

    return sc_kernel


def _tc_body(acc_ref, deg_ref, x_ref, w_ref, r_ref, b_ref, o_ref):
    acc = acc_ref[0, 0:N_NODES, :] + acc_ref[1, 0:N_NODES, :]
    deg = deg_ref[0, 0:N_NODES, 0:1] + deg_ref[1, 0:N_NODES, 0:1]
    scale = 1.0 / jnp.maximum(deg, 1.0)
    o_ref[...] = (
        jnp.dot(acc * scale, w_ref[...], preferred_element_type=jnp.float32)
        + jnp.dot(x_ref[...], r_ref[...], preferred_element_type=jnp.float32)
        + b_ref[...]
    )


def kernel(x, edge_index, edge_attr, weight, root, bias):
    del edge_attr
    e = edge_index.shape[1]
    nb = -(-e // (TILES * B * CH)) * CH
    cap = TILES * B * nb
    src = edge_index[0].astype(jnp.int32)
    dst = edge_index[1].astype(jnp.int32)
    pad = cap - e
    src = jnp.concatenate([src, jnp.zeros((pad,), jnp.int32)])
    dst = jnp.concatenate([dst, jnp.full((pad,), N_PAD - 1, jnp.int32)])
    src = src.reshape(TILES, nb, B)
    dst = dst.reshape(TILES, nb, B)

    acc, deg = _sc_phase(nb)(x, src, dst)

    out = pl.pallas_call(
        _tc_body,
        out_shape=jax.ShapeDtypeStruct((N_NODES, D), jnp.float32),
    )(acc, deg, x, weight[0], root, bias.reshape(1, D))
    return out


# speedup vs baseline: 5.2396x; 1.1417x over previous
import functools

import jax
import jax.numpy as jnp
from jax import lax
from jax.experimental import pallas as pl
from jax.experimental.pallas import tpu as pltpu
from jax.experimental.pallas import tpu_sc as plsc

N_NODES = 10000
D = 128
N_PAD = 10016
ROWS_PER_TILE = N_PAD // 16
TAIL = ROWS_PER_TILE - 4 * 128
DEG_W = 16
B = 128
CH = 8
NUM_CORES = 2
NUM_SUBCORES = 16
TILES = NUM_CORES * NUM_SUBCORES


def _sc_phase(nc):
    mesh = plsc.VectorSubcoreMesh(core_axis_name="c", subcore_axis_name="s")

    @functools.partial(
        pl.kernel,
        out_type=[
            jax.ShapeDtypeStruct((NUM_CORES, N_PAD, D), jnp.float32),
            jax.ShapeDtypeStruct((NUM_CORES, N_PAD, DEG_W), jnp.float32),
        ],
        mesh=mesh,
        scratch_types=[
            pltpu.VMEM((CH, 2, B), jnp.int32),
            pltpu.VMEM((CH, 2, B), jnp.int32),
            pltpu.VMEM((B, D), jnp.float32),
            pltpu.VMEM((B, D), jnp.float32),
            pltpu.VMEM((B, DEG_W), jnp.float32),
            pltpu.SemaphoreType.DMA,
            pltpu.SemaphoreType.DMA,
            pltpu.SemaphoreType.DMA,
            pltpu.VMEM_SHARED((N_PAD, D), jnp.float32),
            pltpu.VMEM_SHARED((N_PAD, DEG_W), jnp.float32),
        ],
        compiler_params=pltpu.CompilerParams(use_tc_tiling_on_sc=False),
    )
    def sc_kernel(x_hbm, sd_hbm, acc_hbm, deg_hbm,
                  sd0, sd1, rows0, rows1, ones_v,
                  gsem0, gsem1, isem, acc_sh, deg_sh):
        cid = lax.axis_index("c")
        sid = lax.axis_index("s")
        wid = cid * NUM_SUBCORES + sid
        rowbase = sid * ROWS_PER_TILE

        zeros16 = jnp.zeros((16,), jnp.float32)
        ones16 = jnp.ones((16,), jnp.float32)
        sds = (sd0, sd1)
        rows = (rows0, rows1)
        gsems = (gsem0, gsem1)

        @pl.loop(0, B)
        def _(i):
            ones_v[i, :] = zeros16
            for k in range(D // 16):
                rows0[i, pl.ds(k * 16, 16)] = zeros16

        for c in range(4):
            pltpu.sync_copy(rows0, acc_sh.at[pl.ds(rowbase + c * B, B)])
            pltpu.sync_copy(ones_v, deg_sh.at[pl.ds(rowbase + c * B, B)])
        pltpu.sync_copy(rows0.at[pl.ds(0, TAIL)],
                        acc_sh.at[pl.ds(rowbase + 4 * B, TAIL)])
        pltpu.sync_copy(ones_v.at[pl.ds(0, TAIL)],
                        deg_sh.at[pl.ds(rowbase + 4 * B, TAIL)])

        @pl.loop(0, B)
        def _(i):
            ones_v[i, :] = ones16

        pltpu.sync_copy(sd_hbm.at[wid, 0], sd0)
        pltpu.async_copy(sd_hbm.at[wid, 1], sd1, isem)

        plsc.subcore_barrier()

        pltpu.async_copy(x_hbm.at[sd0.at[0, 0]], rows0, gsem0)

        def do_chunk(o, par, last):
            sd = sds[par]
            nxt = sds[1 - par]
            for k in range(CH):
                buf, sem = rows[k % 2], gsems[k % 2]
                other, osem = rows[1 - k % 2], gsems[1 - k % 2]
                pltpu.make_async_copy(x_hbm.at[sd.at[k, 0]], buf, sem).wait()
                if k + 1 < CH:
                    pltpu.async_copy(x_hbm.at[sd.at[k + 1, 0]], other, osem)
                elif not last:
                    pltpu.make_async_copy(sd_hbm.at[wid, 0], nxt, isem).wait()
                    pltpu.async_copy(x_hbm.at[nxt.at[0, 0]], other, osem)
                pltpu.sync_copy(buf, acc_sh.at[sd.at[k, 1]], add=True)
                pltpu.sync_copy(ones_v, deg_sh.at[sd.at[k, 1]], add=True)
                if k + 1 == CH and not last:
                    @pl.when(o + 2 < nc)
                    def _():
                        pltpu.async_copy(sd_hbm.at[wid, o + 2], sd, isem)

        if nc % 2 == 1:
            @pl.loop(0, nc - 1, step=2)
            def _(o):
                do_chunk(o, 0, False)
                do_chunk(o + 1, 1, False)

            do_chunk(nc - 1, 0, True)
        else:
            @pl.loop(0, nc - 2, step=2)
            def _(o):
                do_chunk(o, 0, False)
                do_chunk(o + 1, 1, False)

            do_chunk(nc - 2, 0, False)
            do_chunk(nc - 1, 1, True)

        plsc.subcore_barrier()

        for c in range(4):
            pltpu.sync_copy(acc_sh.at[pl.ds(rowbase + c * B, B)], rows0)
            pltpu.sync_copy(rows0, acc_hbm.at[cid, pl.ds(rowbase + c * B, B)])
            pltpu.sync_copy(deg_sh.at[pl.ds(rowbase + c * B, B)], ones_v)
            pltpu.sync_copy(ones_v, deg_hbm.at[cid, pl.ds(rowbase + c * B, B)])
        pltpu.sync_copy(acc_sh.at[pl.ds(rowbase + 4 * B, TAIL)],
                        rows0.at[pl.ds(0, TAIL)])
        pltpu.sync_copy(rows0.at[pl.ds(0, TAIL)],
                        acc_hbm.at[cid, pl.ds(rowbase + 4 * B, TAIL)])
        pltpu.sync_copy(deg_sh.at[pl.ds(rowbase + 4 * B, TAIL)],
                        ones_v.at[pl.ds(0, TAIL)])
        pltpu.sync_copy(ones_v.at[pl.ds(0, TAIL)],
                        deg_hbm.at[cid, pl.ds(rowbase + 4 * B, TAIL)])

    return sc_kernel


def _tc_body(acc_ref, deg_ref, x_ref, w_ref, r_ref, b_ref, o_ref):
    acc = acc_ref[0, 0:N_NODES, :] + acc_ref[1, 0:N_NODES, :]
    deg = deg_ref[0, 0:N_NODES, 0:1] + deg_ref[1, 0:N_NODES, 0:1]
    scale = 1.0 / jnp.maximum(deg, 1.0)
    o_ref[...] = (
        jnp.dot(acc * scale, w_ref[...], preferred_element_type=jnp.float32)
        + jnp.dot(x_ref[...], r_ref[...], preferred_element_type=jnp.float32)
        + b_ref[...]
    )


def kernel(x, edge_index, edge_attr, weight, root, bias):
    del edge_attr
    e = edge_index.shape[1]
    nc = max(2, -(-e // (TILES * B * CH)))
    cap = TILES * B * CH * nc
    src = edge_index[0].astype(jnp.int32)
    dst = edge_index[1].astype(jnp.int32)
    pad = cap - e
    src = jnp.concatenate([src, jnp.zeros((pad,), jnp.int32)])
    dst = jnp.concatenate([dst, jnp.full((pad,), N_PAD - 1, jnp.int32)])
    sd = jnp.stack([src.reshape(TILES, nc, CH, B),
                    dst.reshape(TILES, nc, CH, B)], axis=3)

    acc, deg = _sc_phase(nc)(x, sd)

    out = pl.pallas_call(
        _tc_body,
        out_shape=jax.ShapeDtypeStruct((N_NODES, D), jnp.float32),
    )(acc, deg, x, weight[0], root, bias.reshape(1, D))
    return out


# speedup vs baseline: 5.5069x; 1.0510x over previous
import functools

import jax
import jax.numpy as jnp
from jax import lax
from jax.experimental import pallas as pl
from jax.experimental.pallas import tpu as pltpu
from jax.experimental.pallas import tpu_sc as plsc

N_NODES = 10000
D = 128
N_PAD = 10016
ROWS_PER_TILE = N_PAD // 16
TAIL = ROWS_PER_TILE - 4 * 128
DEG_W = 16
B = 128
CH = 8
NUM_CORES = 2
NUM_SUBCORES = 16
CORE0_FRAC = 0.73


def _sc_phase(k0, k1):
    mesh = plsc.VectorSubcoreMesh(core_axis_name="c", subcore_axis_name="s")

    @functools.partial(
        pl.kernel,
        out_type=[
            jax.ShapeDtypeStruct((NUM_CORES, N_PAD, D), jnp.float32),
            jax.ShapeDtypeStruct((NUM_CORES, N_PAD, DEG_W), jnp.float32),
        ],
        mesh=mesh,
        scratch_types=[
            pltpu.VMEM((CH, 2, B), jnp.int32),
            pltpu.VMEM((CH, 2, B), jnp.int32),
            pltpu.VMEM((B, D), jnp.float32),
            pltpu.VMEM((B, D), jnp.float32),
            pltpu.VMEM((B, DEG_W), jnp.float32),
            pltpu.SemaphoreType.DMA,
            pltpu.SemaphoreType.DMA,
            pltpu.SemaphoreType.DMA,
            pltpu.VMEM_SHARED((N_PAD, D), jnp.float32),
            pltpu.VMEM_SHARED((N_PAD, DEG_W), jnp.float32),
        ],
        compiler_params=pltpu.CompilerParams(use_tc_tiling_on_sc=False),
    )
    def sc_kernel(x_hbm, sd_hbm, acc_hbm, deg_hbm,
                  sd0, sd1, rows0, rows1, ones_v,
                  gsem0, gsem1, isem, acc_sh, deg_sh):
        cid = lax.axis_index("c")
        sid = lax.axis_index("s")
        rowbase = sid * ROWS_PER_TILE

        zeros16 = jnp.zeros((16,), jnp.float32)
        ones16 = jnp.ones((16,), jnp.float32)
        sds = (sd0, sd1)
        rows = (rows0, rows1)
        gsems = (gsem0, gsem1)

        @pl.loop(0, B)
        def _(i):
            ones_v[i, :] = zeros16
            for k in range(D // 16):
                rows0[i, pl.ds(k * 16, 16)] = zeros16

        for c in range(4):
            pltpu.sync_copy(rows0, acc_sh.at[pl.ds(rowbase + c * B, B)])
            pltpu.sync_copy(ones_v, deg_sh.at[pl.ds(rowbase + c * B, B)])
        pltpu.sync_copy(rows0.at[pl.ds(0, TAIL)],
                        acc_sh.at[pl.ds(rowbase + 4 * B, TAIL)])
        pltpu.sync_copy(ones_v.at[pl.ds(0, TAIL)],
                        deg_sh.at[pl.ds(rowbase + 4 * B, TAIL)])

        @pl.loop(0, B)
        def _(i):
            ones_v[i, :] = ones16

        plsc.subcore_barrier()

        def do_chunk(base, o, par, nc, last):
            sd = sds[par]
            nxt = sds[1 - par]
            for k in range(CH):
                buf, sem = rows[k % 2], gsems[k % 2]
                other, osem = rows[1 - k % 2], gsems[1 - k % 2]
                pltpu.make_async_copy(x_hbm.at[sd.at[k, 0]], buf, sem).wait()
                if k + 1 < CH:
                    pltpu.async_copy(x_hbm.at[sd.at[k + 1, 0]], other, osem)
                elif not last:
                    pltpu.make_async_copy(sd_hbm.at[0], nxt, isem).wait()
                    pltpu.async_copy(x_hbm.at[nxt.at[0, 0]], other, osem)
                pltpu.sync_copy(buf, acc_sh.at[sd.at[k, 1]], add=True)
                pltpu.sync_copy(ones_v, deg_sh.at[sd.at[k, 1]], add=True)
                if k + 1 == CH and not last:
                    @pl.when(o + 2 < nc)
                    def _():
                        pltpu.async_copy(sd_hbm.at[base + o + 2], sd, isem)

        def pipeline(nc, base):
            pltpu.sync_copy(sd_hbm.at[base], sd0)
            if nc > 1:
                pltpu.async_copy(sd_hbm.at[base + 1], sd1, isem)
            pltpu.async_copy(x_hbm.at[sd0.at[0, 0]], rows0, gsem0)
            if nc == 1:
                do_chunk(base, 0, 0, nc, True)
            elif nc % 2 == 1:
                @pl.loop(0, nc - 1, step=2)
                def _(o):
                    do_chunk(base, o, 0, nc, False)
                    do_chunk(base, o + 1, 1, nc, False)

                do_chunk(base, nc - 1, 0, nc, True)
            else:
                @pl.loop(0, nc - 2, step=2)
                def _(o):
                    do_chunk(base, o, 0, nc, False)
                    do_chunk(base, o + 1, 1, nc, False)

                do_chunk(base, nc - 2, 0, nc, False)
                do_chunk(base, nc - 1, 1, nc, True)

        @pl.when(cid == 0)
        def _():
            pipeline(k0, sid * k0)

        @pl.when(cid == 1)
        def _():
            pipeline(k1, NUM_SUBCORES * k0 + sid * k1)

        plsc.subcore_barrier()

        for c in range(4):
            pltpu.sync_copy(acc_sh.at[pl.ds(rowbase + c * B, B)], rows0)
            pltpu.sync_copy(rows0, acc_hbm.at[cid, pl.ds(rowbase + c * B, B)])
            pltpu.sync_copy(deg_sh.at[pl.ds(rowbase + c * B, B)], ones_v)
            pltpu.sync_copy(ones_v, deg_hbm.at[cid, pl.ds(rowbase + c * B, B)])
        pltpu.sync_copy(acc_sh.at[pl.ds(rowbase + 4 * B, TAIL)],
                        rows0.at[pl.ds(0, TAIL)])
        pltpu.sync_copy(rows0.at[pl.ds(0, TAIL)],
                        acc_hbm.at[cid, pl.ds(rowbase + 4 * B, TAIL)])
        pltpu.sync_copy(deg_sh.at[pl.ds(rowbase + 4 * B, TAIL)],
                        ones_v.at[pl.ds(0, TAIL)])
        pltpu.sync_copy(ones_v.at[pl.ds(0, TAIL)],
                        deg_hbm.at[cid, pl.ds(rowbase + 4 * B, TAIL)])

    return sc_kernel


def _tc_body(acc_ref, deg_ref, x_ref, w_ref, r_ref, b_ref, o_ref):
    acc = acc_ref[0, 0:N_NODES, :] + acc_ref[1, 0:N_NODES, :]
    deg = deg_ref[0, 0:N_NODES, 0:1] + deg_ref[1, 0:N_NODES, 0:1]
    scale = 1.0 / jnp.maximum(deg, 1.0)
    o_ref[...] = (
        jnp.dot(acc * scale, w_ref[...], preferred_element_type=jnp.float32)
        + jnp.dot(x_ref[...], r_ref[...], preferred_element_type=jnp.float32)
        + b_ref[...]
    )


def kernel(x, edge_index, edge_attr, weight, root, bias):
    del edge_attr
    e = edge_index.shape[1]
    blk = CH * B
    npair = max(2, -(-e // (NUM_SUBCORES * blk)))
    k0 = min(npair - 1, max(1, round(npair * CORE0_FRAC)))
    k1 = npair - k0
    cap = NUM_SUBCORES * npair * blk
    src = edge_index[0].astype(jnp.int32)
    dst = edge_index[1].astype(jnp.int32)
    pad = cap - e
    src = jnp.concatenate([src, jnp.zeros((pad,), jnp.int32)])
    dst = jnp.concatenate([dst, jnp.full((pad,), N_PAD - 1, jnp.int32)])
    nblk = cap // blk
    sd = jnp.stack([src.reshape(nblk, CH, B),
                    dst.reshape(nblk, CH, B)], axis=2)

    acc, deg = _sc_phase(k0, k1)(x, sd)

    out = pl.pallas_call(
        _tc_body,
        out_shape=jax.ShapeDtypeStruct((N_NODES, D), jnp.float32),
    )(acc, deg, x, weight[0], root, bias.reshape(1, D))
    return out


# speedup vs baseline: 5.5277x; 1.0038x over previous
import functools

import jax
import jax.numpy as jnp
from jax import lax
from jax.experimental import pallas as pl
from jax.experimental.pallas import tpu as pltpu
from jax.experimental.pallas import tpu_sc as plsc

N_NODES = 10000
D = 128
N_PAD = 10016
ROWS_PER_TILE = N_PAD // 16
TAIL = ROWS_PER_TILE - 4 * 128
DEG_W = 16
B = 128
CH = 8
NUM_CORES = 2
NUM_SUBCORES = 16
CORE0_FRAC = 0.80


def _sc_phase(k0, k1):
    mesh = plsc.VectorSubcoreMesh(core_axis_name="c", subcore_axis_name="s")

    @functools.partial(
        pl.kernel,
        out_type=[
            jax.ShapeDtypeStruct((NUM_CORES, N_PAD, D), jnp.float32),
            jax.ShapeDtypeStruct((NUM_CORES, N_PAD, DEG_W), jnp.float32),
        ],
        mesh=mesh,
        scratch_types=[
            pltpu.VMEM((CH, 2, B), jnp.int32),
            pltpu.VMEM((CH, 2, B), jnp.int32),
            pltpu.VMEM((B, D), jnp.float32),
            pltpu.VMEM((B, D), jnp.float32),
            pltpu.VMEM((B, DEG_W), jnp.float32),
            pltpu.SemaphoreType.DMA,
            pltpu.SemaphoreType.DMA,
            pltpu.SemaphoreType.DMA,
            pltpu.VMEM_SHARED((N_PAD, D), jnp.float32),
            pltpu.VMEM_SHARED((N_PAD, DEG_W), jnp.float32),
        ],
        compiler_params=pltpu.CompilerParams(use_tc_tiling_on_sc=False),
    )
    def sc_kernel(x_hbm, sd_hbm, acc_hbm, deg_hbm,
                  sd0, sd1, rows0, rows1, ones_v,
                  gsem0, gsem1, isem, acc_sh, deg_sh):
        cid = lax.axis_index("c")
        sid = lax.axis_index("s")
        rowbase = sid * ROWS_PER_TILE

        zeros16 = jnp.zeros((16,), jnp.float32)
        ones16 = jnp.ones((16,), jnp.float32)
        sds = (sd0, sd1)
        rows = (rows0, rows1)
        gsems = (gsem0, gsem1)

        @pl.loop(0, B)
        def _(i):
            ones_v[i, :] = zeros16
            for k in range(D // 16):
                rows0[i, pl.ds(k * 16, 16)] = zeros16

        for c in range(4):
            pltpu.sync_copy(rows0, acc_sh.at[pl.ds(rowbase + c * B, B)])
            pltpu.sync_copy(ones_v, deg_sh.at[pl.ds(rowbase + c * B, B)])
        pltpu.sync_copy(rows0.at[pl.ds(0, TAIL)],
                        acc_sh.at[pl.ds(rowbase + 4 * B, TAIL)])
        pltpu.sync_copy(ones_v.at[pl.ds(0, TAIL)],
                        deg_sh.at[pl.ds(rowbase + 4 * B, TAIL)])

        @pl.loop(0, B)
        def _(i):
            ones_v[i, :] = ones16

        plsc.subcore_barrier()

        def do_chunk(base, o, par, nc, last):
            sd = sds[par]
            nxt = sds[1 - par]
            for k in range(CH):
                buf, sem = rows[k % 2], gsems[k % 2]
                other, osem = rows[1 - k % 2], gsems[1 - k % 2]
                pltpu.make_async_copy(x_hbm.at[sd.at[k, 0]], buf, sem).wait()
                if k + 1 < CH:
                    pltpu.async_copy(x_hbm.at[sd.at[k + 1, 0]], other, osem)
                elif not last:
                    pltpu.make_async_copy(sd_hbm.at[0], nxt, isem).wait()
                    pltpu.async_copy(x_hbm.at[nxt.at[0, 0]], other, osem)
                pltpu.sync_copy(buf, acc_sh.at[sd.at[k, 1]], add=True)
                pltpu.sync_copy(ones_v, deg_sh.at[sd.at[k, 1]], add=True)
                if k + 1 == CH and not last:
                    @pl.when(o + 2 < nc)
                    def _():
                        pltpu.async_copy(sd_hbm.at[base + o + 2], sd, isem)

        def pipeline(nc, base):
            pltpu.sync_copy(sd_hbm.at[base], sd0)
            if nc > 1:
                pltpu.async_copy(sd_hbm.at[base + 1], sd1, isem)
            pltpu.async_copy(x_hbm.at[sd0.at[0, 0]], rows0, gsem0)
            if nc == 1:
                do_chunk(base, 0, 0, nc, True)
            elif nc % 2 == 1:
                @pl.loop(0, nc - 1, step=2)
                def _(o):
                    do_chunk(base, o, 0, nc, False)
                    do_chunk(base, o + 1, 1, nc, False)

                do_chunk(base, nc - 1, 0, nc, True)
            else:
                @pl.loop(0, nc - 2, step=2)
                def _(o):
                    do_chunk(base, o, 0, nc, False)
                    do_chunk(base, o + 1, 1, nc, False)

                do_chunk(base, nc - 2, 0, nc, False)
                do_chunk(base, nc - 1, 1, nc, True)

        @pl.when(cid == 0)
        def _():
            pipeline(k0, sid * k0)

        @pl.when(cid == 1)
        def _():
            pipeline(k1, NUM_SUBCORES * k0 + sid * k1)

        plsc.subcore_barrier()

        for c in range(4):
            pltpu.sync_copy(acc_sh.at[pl.ds(rowbase + c * B, B)], rows0)
            pltpu.sync_copy(rows0, acc_hbm.at[cid, pl.ds(rowbase + c * B, B)])
            pltpu.sync_copy(deg_sh.at[pl.ds(rowbase + c * B, B)], ones_v)
            pltpu.sync_copy(ones_v, deg_hbm.at[cid, pl.ds(rowbase + c * B, B)])
        pltpu.sync_copy(acc_sh.at[pl.ds(rowbase + 4 * B, TAIL)],
                        rows0.at[pl.ds(0, TAIL)])
        pltpu.sync_copy(rows0.at[pl.ds(0, TAIL)],
                        acc_hbm.at[cid, pl.ds(rowbase + 4 * B, TAIL)])
        pltpu.sync_copy(deg_sh.at[pl.ds(rowbase + 4 * B, TAIL)],
                        ones_v.at[pl.ds(0, TAIL)])
        pltpu.sync_copy(ones_v.at[pl.ds(0, TAIL)],
                        deg_hbm.at[cid, pl.ds(rowbase + 4 * B, TAIL)])

    return sc_kernel


def _tc_body(acc_ref, deg_ref, x_ref, w_ref, r_ref, b_ref, o_ref):
    acc = acc_ref[0, 0:N_NODES, :] + acc_ref[1, 0:N_NODES, :]
    deg = deg_ref[0, 0:N_NODES, 0:1] + deg_ref[1, 0:N_NODES, 0:1]
    scale = 1.0 / jnp.maximum(deg, 1.0)
    o_ref[...] = (
        jnp.dot(acc * scale, w_ref[...], preferred_element_type=jnp.float32)
        + jnp.dot(x_ref[...], r_ref[...], preferred_element_type=jnp.float32)
        + b_ref[...]
    )


def kernel(x, edge_index, edge_attr, weight, root, bias):
    del edge_attr
    e = edge_index.shape[1]
    blk = CH * B
    npair = max(2, -(-e // (NUM_SUBCORES * blk)))
    k0 = min(npair - 1, max(1, round(npair * CORE0_FRAC)))
    k1 = npair - k0
    cap = NUM_SUBCORES * npair * blk
    src = edge_index[0].astype(jnp.int32)
    dst = edge_index[1].astype(jnp.int32)
    pad = cap - e
    src = jnp.concatenate([src, jnp.zeros((pad,), jnp.int32)])
    dst = jnp.concatenate([dst, jnp.full((pad,), N_PAD - 1, jnp.int32)])
    nblk = cap // blk
    sd = jnp.stack([src.reshape(nblk, CH, B),
                    dst.reshape(nblk, CH, B)], axis=2)

    acc, deg = _sc_phase(k0, k1)(x, sd)

    out = pl.pallas_call(
        _tc_body,
        out_shape=jax.ShapeDtypeStruct((N_NODES, D), jnp.float32),
    )(acc, deg, x, weight[0], root, bias.reshape(1, D))
    return out


# speedup vs baseline: 12.9606x; 2.3447x over previous
import functools

import jax
import jax.numpy as jnp
from jax import lax
from jax.experimental import pallas as pl
from jax.experimental.pallas import tpu as pltpu
from jax.experimental.pallas import tpu_sc as plsc

N_NODES = 10000
D = 128
N_PAD = 10016
ROWS_PER_TILE = N_PAD // 16
TAIL = ROWS_PER_TILE - 4 * 128
DEG_W = 16
B = 128
CH = 8
NUM_CORES = 2
NUM_SUBCORES = 16
CORE0_FRAC = 0.50


def _sc_phase(k0, k1):
    mesh = plsc.VectorSubcoreMesh(core_axis_name="c", subcore_axis_name="s")

    @functools.partial(
        pl.kernel,
        out_type=[
            jax.ShapeDtypeStruct((NUM_CORES, N_PAD, D), jnp.float32),
            jax.ShapeDtypeStruct((NUM_CORES, N_PAD, DEG_W), jnp.float32),
        ],
        mesh=mesh,
        scratch_types=[
            pltpu.VMEM((CH, 2, B), jnp.int32),
            pltpu.VMEM((CH, 2, B), jnp.int32),
            pltpu.VMEM((B, D), jnp.float32),
            pltpu.VMEM((B, D), jnp.float32),
            pltpu.VMEM((B, DEG_W), jnp.float32),
            pltpu.SemaphoreType.DMA,
            pltpu.SemaphoreType.DMA,
            pltpu.SemaphoreType.DMA,
            pltpu.VMEM_SHARED((N_PAD, D), jnp.float32),
            pltpu.VMEM_SHARED((N_PAD, DEG_W), jnp.float32),
        ],
        compiler_params=pltpu.CompilerParams(use_tc_tiling_on_sc=False),
    )
    def sc_kernel(x_hbm, sd_hbm, acc_hbm, deg_hbm,
                  sd0, sd1, rows0, rows1, ones_v,
                  gsem0, gsem1, isem, acc_sh, deg_sh):
        cid = lax.axis_index("c")
        sid = lax.axis_index("s")
        rowbase = sid * ROWS_PER_TILE

        zeros16 = jnp.zeros((16,), jnp.float32)
        ones16 = jnp.ones((16,), jnp.float32)
        sds = (sd0, sd1)
        rows = (rows0, rows1)
        gsems = (gsem0, gsem1)

        @pl.loop(0, B)
        def _(i):
            ones_v[i, :] = zeros16
            for k in range(D // 16):
                rows0[i, pl.ds(k * 16, 16)] = zeros16

        for c in range(4):
            pltpu.sync_copy(rows0, acc_sh.at[pl.ds(rowbase + c * B, B)])
            pltpu.sync_copy(ones_v, deg_sh.at[pl.ds(rowbase + c * B, B)])
        pltpu.sync_copy(rows0.at[pl.ds(0, TAIL)],
                        acc_sh.at[pl.ds(rowbase + 4 * B, TAIL)])
        pltpu.sync_copy(ones_v.at[pl.ds(0, TAIL)],
                        deg_sh.at[pl.ds(rowbase + 4 * B, TAIL)])

        @pl.loop(0, B)
        def _(i):
            ones_v[i, :] = ones16

        plsc.subcore_barrier()

        def do_chunk(base, o, par, nc, last):
            sd = sds[par]
            nxt = sds[1 - par]
            for k in range(CH):
                buf, sem = rows[k % 2], gsems[k % 2]
                other, osem = rows[1 - k % 2], gsems[1 - k % 2]
                pltpu.make_async_copy(x_hbm.at[sd.at[k, 0]], buf, sem).wait()
                if k + 1 < CH:
                    pltpu.async_copy(x_hbm.at[sd.at[k + 1, 0]], other, osem)
                elif not last:
                    pltpu.make_async_copy(sd_hbm.at[0], nxt, isem).wait()
                    pltpu.async_copy(x_hbm.at[nxt.at[0, 0]], other, osem)
                pltpu.sync_copy(buf, acc_sh.at[sd.at[k, 1]], add=True)
                pltpu.sync_copy(ones_v, deg_sh.at[sd.at[k, 1]], add=True)
                if k + 1 == CH and not last:
                    @pl.when(o + 2 < nc)
                    def _():
                        pltpu.async_copy(sd_hbm.at[base + o + 2], sd, isem)

        def pipeline(nc, base):
            pltpu.sync_copy(sd_hbm.at[base], sd0)
            if nc > 1:
                pltpu.async_copy(sd_hbm.at[base + 1], sd1, isem)
            pltpu.async_copy(x_hbm.at[sd0.at[0, 0]], rows0, gsem0)
            if nc == 1:
                do_chunk(base, 0, 0, nc, True)
            elif nc % 2 == 1:
                @pl.loop(0, nc - 1, step=2)
                def _(o):
                    do_chunk(base, o, 0, nc, False)
                    do_chunk(base, o + 1, 1, nc, False)

                do_chunk(base, nc - 1, 0, nc, True)
            else:
                @pl.loop(0, nc - 2, step=2)
                def _(o):
                    do_chunk(base, o, 0, nc, False)
                    do_chunk(base, o + 1, 1, nc, False)

                do_chunk(base, nc - 2, 0, nc, False)
                do_chunk(base, nc - 1, 1, nc, True)

        @pl.when(cid == 0)
        def _():
            pipeline(k0, sid * k0)

        @pl.when(cid == 1)
        def _():
            pipeline(k1, NUM_SUBCORES * k0 + sid * k1)

        plsc.subcore_barrier()

        for c in range(4):
            pltpu.sync_copy(acc_sh.at[pl.ds(rowbase + c * B, B)], rows0)
            pltpu.sync_copy(rows0, acc_hbm.at[cid, pl.ds(rowbase + c * B, B)])
            pltpu.sync_copy(deg_sh.at[pl.ds(rowbase + c * B, B)], ones_v)
            pltpu.sync_copy(ones_v, deg_hbm.at[cid, pl.ds(rowbase + c * B, B)])
        pltpu.sync_copy(acc_sh.at[pl.ds(rowbase + 4 * B, TAIL)],
                        rows0.at[pl.ds(0, TAIL)])
        pltpu.sync_copy(rows0.at[pl.ds(0, TAIL)],
                        acc_hbm.at[cid, pl.ds(rowbase + 4 * B, TAIL)])
        pltpu.sync_copy(deg_sh.at[pl.ds(rowbase + 4 * B, TAIL)],
                        ones_v.at[pl.ds(0, TAIL)])
        pltpu.sync_copy(ones_v.at[pl.ds(0, TAIL)],
                        deg_hbm.at[cid, pl.ds(rowbase + 4 * B, TAIL)])

    return sc_kernel


def _tc_body(acc_ref, deg_ref, x_ref, w_ref, r_ref, b_ref, o_ref):
    acc = acc_ref[0, 0:N_NODES, :] + acc_ref[1, 0:N_NODES, :]
    deg = deg_ref[0, 0:N_NODES, 0:1] + deg_ref[1, 0:N_NODES, 0:1]
    scale = 1.0 / jnp.maximum(deg, 1.0)
    o_ref[...] = (
        jnp.dot(acc * scale, w_ref[...], preferred_element_type=jnp.float32)
        + jnp.dot(x_ref[...], r_ref[...], preferred_element_type=jnp.float32)
        + b_ref[...]
    )


def kernel(x, edge_index, edge_attr, weight, root, bias):
    del edge_attr
    e = edge_index.shape[1]
    blk = CH * B
    npair = max(2, -(-e // (NUM_SUBCORES * blk)))
    k0 = min(npair - 1, max(1, round(npair * CORE0_FRAC)))
    k1 = npair - k0
    cap = NUM_SUBCORES * npair * blk
    src = edge_index[0].astype(jnp.int32)
    dst = edge_index[1].astype(jnp.int32)
    pad = cap - e
    cyc = jnp.arange(pad, dtype=jnp.int32) % (N_PAD - N_NODES)
    src = jnp.concatenate([src, cyc])
    dst = jnp.concatenate([dst, N_NODES + cyc])
    nblk = cap // blk
    sd = jnp.stack([src.reshape(nblk, CH, B),
                    dst.reshape(nblk, CH, B)], axis=2)

    acc, deg = _sc_phase(k0, k1)(x, sd)

    out = pl.pallas_call(
        _tc_body,
        out_shape=jax.ShapeDtypeStruct((N_NODES, D), jnp.float32),
    )(acc, deg, x, weight[0], root, bias.reshape(1, D))
    return out


# speedup vs baseline: 13.0912x; 1.0101x over previous
import functools

import jax
import jax.numpy as jnp
from jax import lax
from jax.experimental import pallas as pl
from jax.experimental.pallas import tpu as pltpu
from jax.experimental.pallas import tpu_sc as plsc

N_NODES = 10000
D = 128
N_PAD = 10016
ROWS_PER_TILE = N_PAD // 16
TAIL = ROWS_PER_TILE - 4 * 128
DEG_W = 16
B = 128
CH = 8
NUM_CORES = 2
NUM_SUBCORES = 16
CORE0_FRAC = 0.50


def _sc_phase(k0, k1):
    mesh = plsc.VectorSubcoreMesh(core_axis_name="c", subcore_axis_name="s")

    @functools.partial(
        pl.kernel,
        out_type=[
            jax.ShapeDtypeStruct((NUM_CORES, N_PAD, D), jnp.float32),
            jax.ShapeDtypeStruct((NUM_CORES, N_PAD, DEG_W), jnp.float32),
        ],
        mesh=mesh,
        scratch_types=[
            pltpu.VMEM((2, CH, B), jnp.int32),
            pltpu.VMEM((2, CH, B), jnp.int32),
            pltpu.VMEM((B, D), jnp.float32),
            pltpu.VMEM((B, D), jnp.float32),
            pltpu.VMEM((B, DEG_W), jnp.float32),
            pltpu.SemaphoreType.DMA,
            pltpu.SemaphoreType.DMA,
            pltpu.SemaphoreType.DMA,
            pltpu.VMEM_SHARED((N_PAD, D), jnp.float32),
            pltpu.VMEM_SHARED((N_PAD, DEG_W), jnp.float32),
        ],
        compiler_params=pltpu.CompilerParams(use_tc_tiling_on_sc=False),
    )
    def sc_kernel(x_hbm, s_hbm, d_hbm, acc_hbm, deg_hbm,
                  sd0, sd1, rows0, rows1, ones_v,
                  gsem0, gsem1, isem, acc_sh, deg_sh):
        cid = lax.axis_index("c")
        sid = lax.axis_index("s")
        rowbase = sid * ROWS_PER_TILE

        zeros16 = jnp.zeros((16,), jnp.float32)
        ones16 = jnp.ones((16,), jnp.float32)
        sds = (sd0, sd1)
        rows = (rows0, rows1)
        gsems = (gsem0, gsem1)

        @pl.loop(0, B)
        def _(i):
            ones_v[i, :] = zeros16
            for k in range(D // 16):
                rows0[i, pl.ds(k * 16, 16)] = zeros16

        for c in range(4):
            pltpu.sync_copy(rows0, acc_sh.at[pl.ds(rowbase + c * B, B)])
            pltpu.sync_copy(ones_v, deg_sh.at[pl.ds(rowbase + c * B, B)])
        pltpu.sync_copy(rows0.at[pl.ds(0, TAIL)],
                        acc_sh.at[pl.ds(rowbase + 4 * B, TAIL)])
        pltpu.sync_copy(ones_v.at[pl.ds(0, TAIL)],
                        deg_sh.at[pl.ds(rowbase + 4 * B, TAIL)])

        @pl.loop(0, B)
        def _(i):
            ones_v[i, :] = ones16

        plsc.subcore_barrier()

        def do_chunk(base, o, par, nc, last):
            sd = sds[par]
            nxt = sds[1 - par]
            for k in range(CH):
                buf, sem = rows[k % 2], gsems[k % 2]
                other, osem = rows[1 - k % 2], gsems[1 - k % 2]
                pltpu.make_async_copy(x_hbm.at[sd.at[0, k]], buf, sem).wait()
                if k + 1 < CH:
                    pltpu.async_copy(x_hbm.at[sd.at[0, k + 1]], other, osem)
                elif not last:
                    pltpu.make_async_copy(s_hbm.at[0], nxt.at[0], isem).wait()
                    pltpu.make_async_copy(d_hbm.at[0], nxt.at[1], isem).wait()
                    pltpu.async_copy(x_hbm.at[nxt.at[0, 0]], other, osem)
                pltpu.sync_copy(buf, acc_sh.at[sd.at[1, k]], add=True)
                pltpu.sync_copy(ones_v, deg_sh.at[sd.at[1, k]], add=True)
                if k + 1 == CH and not last:
                    @pl.when(o + 2 < nc)
                    def _():
                        pltpu.async_copy(s_hbm.at[base + o + 2], sd.at[0], isem)
                        pltpu.async_copy(d_hbm.at[base + o + 2], sd.at[1], isem)

        def pipeline(nc, base):
            pltpu.sync_copy(s_hbm.at[base], sd0.at[0])
            pltpu.sync_copy(d_hbm.at[base], sd0.at[1])
            if nc > 1:
                pltpu.async_copy(s_hbm.at[base + 1], sd1.at[0], isem)
                pltpu.async_copy(d_hbm.at[base + 1], sd1.at[1], isem)
            pltpu.async_copy(x_hbm.at[sd0.at[0, 0]], rows0, gsem0)
            if nc == 1:
                do_chunk(base, 0, 0, nc, True)
            elif nc % 2 == 1:
                @pl.loop(0, nc - 1, step=2)
                def _(o):
                    do_chunk(base, o, 0, nc, False)
                    do_chunk(base, o + 1, 1, nc, False)

                do_chunk(base, nc - 1, 0, nc, True)
            else:
                @pl.loop(0, nc - 2, step=2)
                def _(o):
                    do_chunk(base, o, 0, nc, False)
                    do_chunk(base, o + 1, 1, nc, False)

                do_chunk(base, nc - 2, 0, nc, False)
                do_chunk(base, nc - 1, 1, nc, True)

        @pl.when(cid == 0)
        def _():
            pipeline(k0, sid * k0)

        @pl.when(cid == 1)
        def _():
            pipeline(k1, NUM_SUBCORES * k0 + sid * k1)

        plsc.subcore_barrier()

        for c in range(4):
            pltpu.sync_copy(acc_sh.at[pl.ds(rowbase + c * B, B)], rows0)
            pltpu.sync_copy(rows0, acc_hbm.at[cid, pl.ds(rowbase + c * B, B)])
            pltpu.sync_copy(deg_sh.at[pl.ds(rowbase + c * B, B)], ones_v)
            pltpu.sync_copy(ones_v, deg_hbm.at[cid, pl.ds(rowbase + c * B, B)])
        pltpu.sync_copy(acc_sh.at[pl.ds(rowbase + 4 * B, TAIL)],
                        rows0.at[pl.ds(0, TAIL)])
        pltpu.sync_copy(rows0.at[pl.ds(0, TAIL)],
                        acc_hbm.at[cid, pl.ds(rowbase + 4 * B, TAIL)])
        pltpu.sync_copy(deg_sh.at[pl.ds(rowbase + 4 * B, TAIL)],
                        ones_v.at[pl.ds(0, TAIL)])
        pltpu.sync_copy(ones_v.at[pl.ds(0, TAIL)],
                        deg_hbm.at[cid, pl.ds(rowbase + 4 * B, TAIL)])

    return sc_kernel


def _xr_body(x_ref, r_ref, b_ref, o_ref):
    o_ref[...] = (
        jnp.dot(x_ref[...], r_ref[...], preferred_element_type=jnp.float32)
        + b_ref[...]
    )


def _tc_body(acc_ref, deg_ref, xr_ref, w_ref, o_ref):
    acc = acc_ref[0, 0:N_NODES, :] + acc_ref[1, 0:N_NODES, :]
    deg = deg_ref[0, 0:N_NODES, 0:1] + deg_ref[1, 0:N_NODES, 0:1]
    scale = 1.0 / jnp.maximum(deg, 1.0)
    o_ref[...] = (
        jnp.dot(acc * scale, w_ref[...], preferred_element_type=jnp.float32)
        + xr_ref[...]
    )


def kernel(x, edge_index, edge_attr, weight, root, bias):
    del edge_attr
    e = edge_index.shape[1]
    blk = CH * B
    npair = max(2, -(-e // (NUM_SUBCORES * blk)))
    k0 = min(npair - 1, max(1, round(npair * CORE0_FRAC)))
    k1 = npair - k0
    cap = NUM_SUBCORES * npair * blk
    src = edge_index[0].astype(jnp.int32)
    dst = edge_index[1].astype(jnp.int32)
    pad = cap - e
    cyc = jnp.arange(pad, dtype=jnp.int32) % (N_PAD - N_NODES)
    nblk = cap // blk
    s_idx = jnp.concatenate([src, cyc]).reshape(nblk, CH, B)
    d_idx = jnp.concatenate([dst, N_NODES + cyc]).reshape(nblk, CH, B)

    xr = pl.pallas_call(
        _xr_body,
        out_shape=jax.ShapeDtypeStruct((N_NODES, D), jnp.float32),
    )(x, root, bias.reshape(1, D))

    acc, deg = _sc_phase(k0, k1)(x, s_idx, d_idx)

    out = pl.pallas_call(
        _tc_body,
        out_shape=jax.ShapeDtypeStruct((N_NODES, D), jnp.float32),
    )(acc, deg, xr, weight[0])
    return out


# speedup vs baseline: 13.2454x; 1.0118x over previous
import functools

import jax
import jax.numpy as jnp
from jax import lax
from jax.experimental import pallas as pl
from jax.experimental.pallas import tpu as pltpu
from jax.experimental.pallas import tpu_sc as plsc

N_NODES = 10000
D = 128
N_PAD = 10016
ROWS_PER_TILE = N_PAD // 16
TAIL = ROWS_PER_TILE - 4 * 128
DEG_W = 16
B = 128
CH = 8
NUM_CORES = 2
NUM_SUBCORES = 16
CORE0_FRAC = 0.50


def _sc_phase(k0, k1):
    mesh = plsc.VectorSubcoreMesh(core_axis_name="c", subcore_axis_name="s")

    @functools.partial(
        pl.kernel,
        out_type=[
            jax.ShapeDtypeStruct((NUM_CORES, N_PAD, D), jnp.float32),
            jax.ShapeDtypeStruct((NUM_CORES, N_PAD, DEG_W), jnp.float32),
        ],
        mesh=mesh,
        scratch_types=[
            pltpu.VMEM((CH, 2, B), jnp.int32),
            pltpu.VMEM((CH, 2, B), jnp.int32),
            pltpu.VMEM((B, D), jnp.float32),
            pltpu.VMEM((B, D), jnp.float32),
            pltpu.VMEM((B, DEG_W), jnp.float32),
            pltpu.SemaphoreType.DMA,
            pltpu.SemaphoreType.DMA,
            pltpu.SemaphoreType.DMA,
            pltpu.VMEM_SHARED((N_PAD, D), jnp.float32),
            pltpu.VMEM_SHARED((N_PAD, DEG_W), jnp.float32),
        ],
        compiler_params=pltpu.CompilerParams(use_tc_tiling_on_sc=False),
    )
    def sc_kernel(x_hbm, sd_hbm, acc_hbm, deg_hbm,
                  sd0, sd1, rows0, rows1, ones_v,
                  gsem0, gsem1, isem, acc_sh, deg_sh):
        cid = lax.axis_index("c")
        sid = lax.axis_index("s")
        rowbase = sid * ROWS_PER_TILE

        zeros16 = jnp.zeros((16,), jnp.float32)
        ones16 = jnp.ones((16,), jnp.float32)
        sds = (sd0, sd1)
        rows = (rows0, rows1)
        gsems = (gsem0, gsem1)

        @pl.loop(0, B)
        def _(i):
            ones_v[i, :] = zeros16
            for k in range(D // 16):
                rows0[i, pl.ds(k * 16, 16)] = zeros16

        for c in range(4):
            pltpu.sync_copy(rows0, acc_sh.at[pl.ds(rowbase + c * B, B)])
            pltpu.sync_copy(ones_v, deg_sh.at[pl.ds(rowbase + c * B, B)])
        pltpu.sync_copy(rows0.at[pl.ds(0, TAIL)],
                        acc_sh.at[pl.ds(rowbase + 4 * B, TAIL)])
        pltpu.sync_copy(ones_v.at[pl.ds(0, TAIL)],
                        deg_sh.at[pl.ds(rowbase + 4 * B, TAIL)])

        @pl.loop(0, B)
        def _(i):
            ones_v[i, :] = ones16

        plsc.subcore_barrier()

        def do_chunk(base, o, par, nc, last):
            sd = sds[par]
            nxt = sds[1 - par]
            for k in range(CH):
                buf, sem = rows[k % 2], gsems[k % 2]
                other, osem = rows[1 - k % 2], gsems[1 - k % 2]
                pltpu.make_async_copy(x_hbm.at[sd.at[k, 0]], buf, sem).wait()
                if k + 1 < CH:
                    pltpu.async_copy(x_hbm.at[sd.at[k + 1, 0]], other, osem)
                elif not last:
                    pltpu.make_async_copy(sd_hbm.at[0], nxt, isem).wait()
                    pltpu.async_copy(x_hbm.at[nxt.at[0, 0]], other, osem)
                pltpu.sync_copy(buf, acc_sh.at[sd.at[k, 1]], add=True)
                pltpu.sync_copy(ones_v, deg_sh.at[sd.at[k, 1]], add=True)
                if k + 1 == CH and not last:
                    @pl.when(o + 2 < nc)
                    def _():
                        pltpu.async_copy(sd_hbm.at[base + o + 2], sd, isem)

        def pipeline(nc, base):
            pltpu.sync_copy(sd_hbm.at[base], sd0)
            if nc > 1:
                pltpu.async_copy(sd_hbm.at[base + 1], sd1, isem)
            pltpu.async_copy(x_hbm.at[sd0.at[0, 0]], rows0, gsem0)
            if nc == 1:
                do_chunk(base, 0, 0, nc, True)
            elif nc % 2 == 1:
                @pl.loop(0, nc - 1, step=2)
                def _(o):
                    do_chunk(base, o, 0, nc, False)
                    do_chunk(base, o + 1, 1, nc, False)

                do_chunk(base, nc - 1, 0, nc, True)
            else:
                @pl.loop(0, nc - 2, step=2)
                def _(o):
                    do_chunk(base, o, 0, nc, False)
                    do_chunk(base, o + 1, 1, nc, False)

                do_chunk(base, nc - 2, 0, nc, False)
                do_chunk(base, nc - 1, 1, nc, True)

        @pl.when(cid == 0)
        def _():
            pipeline(k0, sid * k0)

        @pl.when(cid == 1)
        def _():
            pipeline(k1, NUM_SUBCORES * k0 + sid * k1)

        plsc.subcore_barrier()

        for c in range(4):
            pltpu.sync_copy(acc_sh.at[pl.ds(rowbase + c * B, B)], rows0)
            pltpu.sync_copy(rows0, acc_hbm.at[cid, pl.ds(rowbase + c * B, B)])
            pltpu.sync_copy(deg_sh.at[pl.ds(rowbase + c * B, B)], ones_v)
            pltpu.sync_copy(ones_v, deg_hbm.at[cid, pl.ds(rowbase + c * B, B)])
        pltpu.sync_copy(acc_sh.at[pl.ds(rowbase + 4 * B, TAIL)],
                        rows0.at[pl.ds(0, TAIL)])
        pltpu.sync_copy(rows0.at[pl.ds(0, TAIL)],
                        acc_hbm.at[cid, pl.ds(rowbase + 4 * B, TAIL)])
        pltpu.sync_copy(deg_sh.at[pl.ds(rowbase + 4 * B, TAIL)],
                        ones_v.at[pl.ds(0, TAIL)])
        pltpu.sync_copy(ones_v.at[pl.ds(0, TAIL)],
                        deg_hbm.at[cid, pl.ds(rowbase + 4 * B, TAIL)])

    return sc_kernel


def _xr_body(x_ref, r_ref, b_ref, o_ref):
    o_ref[...] = (
        jnp.dot(x_ref[...], r_ref[...], preferred_element_type=jnp.float32)
        + b_ref[...]
    )


def _tc_body(acc_ref, deg_ref, xr_ref, w_ref, o_ref):
    acc = acc_ref[0, 0:N_NODES, :] + acc_ref[1, 0:N_NODES, :]
    deg = deg_ref[0, 0:N_NODES, 0:1] + deg_ref[1, 0:N_NODES, 0:1]
    scale = 1.0 / jnp.maximum(deg, 1.0)
    o_ref[...] = (
        jnp.dot(acc * scale, w_ref[...], preferred_element_type=jnp.float32)
        + xr_ref[...]
    )


def kernel(x, edge_index, edge_attr, weight, root, bias):
    del edge_attr
    e = edge_index.shape[1]
    blk = CH * B
    npair = max(2, -(-e // (NUM_SUBCORES * blk)))
    k0 = min(npair - 1, max(1, round(npair * CORE0_FRAC)))
    k1 = npair - k0
    cap = NUM_SUBCORES * npair * blk
    pad = cap - e
    cyc = jnp.arange(pad, dtype=jnp.int32) % (N_PAD - N_NODES)
    nblk = cap // blk
    sd_real = edge_index.astype(jnp.int32).reshape(2, e // B, B)
    sd_pad = jnp.stack([cyc, N_NODES + cyc]).reshape(2, pad // B, B)
    sd = jnp.concatenate([sd_real.transpose(1, 0, 2),
                          sd_pad.transpose(1, 0, 2)])
    sd = sd.reshape(nblk, CH, 2, B)

    xr = pl.pallas_call(
        _xr_body,
        out_shape=jax.ShapeDtypeStruct((N_NODES, D), jnp.float32),
    )(x, root, bias.reshape(1, D))

    acc, deg = _sc_phase(k0, k1)(x, sd)

    out = pl.pallas_call(
        _tc_body,
        out_shape=jax.ShapeDtypeStruct((N_NODES, D), jnp.float32),
    )(acc, deg, xr, weight[0])
    return out


# speedup vs baseline: 13.7476x; 1.0379x over previous
import functools

import jax
import jax.numpy as jnp
from jax import lax
from jax.experimental import pallas as pl
from jax.experimental.pallas import tpu as pltpu
from jax.experimental.pallas import tpu_sc as plsc

N_NODES = 10000
D = 128
N_PAD = 10016
ROWS_PER_TILE = N_PAD // 16
TAIL = ROWS_PER_TILE - 4 * 128
DEG_W = 16
B = 128
CH = 8
NUM_CORES = 2
NUM_SUBCORES = 16
CORE0_FRAC = 0.50


def _sc_phase(k0, k1):
    mesh = plsc.VectorSubcoreMesh(core_axis_name="c", subcore_axis_name="s")

    @functools.partial(
        pl.kernel,
        out_type=[
            jax.ShapeDtypeStruct((NUM_CORES, N_PAD, D), jnp.float32),
            jax.ShapeDtypeStruct((NUM_CORES, N_PAD, DEG_W), jnp.float32),
        ],
        mesh=mesh,
        scratch_types=[
            pltpu.VMEM((CH, 2, B), jnp.int32),
            pltpu.VMEM((CH, 2, B), jnp.int32),
            pltpu.VMEM((B, D), jnp.float32),
            pltpu.VMEM((B, D), jnp.float32),
            pltpu.VMEM((B, DEG_W), jnp.float32),
            pltpu.SemaphoreType.DMA,
            pltpu.SemaphoreType.DMA,
            pltpu.SemaphoreType.DMA,
            pltpu.VMEM_SHARED((N_PAD, D), jnp.float32),
            pltpu.VMEM_SHARED((N_PAD, DEG_W), jnp.float32),
        ],
        compiler_params=pltpu.CompilerParams(use_tc_tiling_on_sc=False),
    )
    def sc_kernel(x_hbm, sd_hbm, acc_hbm, deg_hbm,
                  sd0, sd1, rows0, rows1, ones_v,
                  gsem0, gsem1, isem, acc_sh, deg_sh):
        cid = lax.axis_index("c")
        sid = lax.axis_index("s")
        rowbase = sid * ROWS_PER_TILE

        zeros16 = jnp.zeros((16,), jnp.float32)
        ones16 = jnp.ones((16,), jnp.float32)
        sds = (sd0, sd1)
        rows = (rows0, rows1)
        gsems = (gsem0, gsem1)

        @pl.loop(0, B)
        def _(i):
            ones_v[i, :] = zeros16
            for k in range(D // 16):
                rows0[i, pl.ds(k * 16, 16)] = zeros16

        for c in range(4):
            pltpu.sync_copy(rows0, acc_sh.at[pl.ds(rowbase + c * B, B)])
            pltpu.sync_copy(ones_v, deg_sh.at[pl.ds(rowbase + c * B, B)])
        pltpu.sync_copy(rows0.at[pl.ds(0, TAIL)],
                        acc_sh.at[pl.ds(rowbase + 4 * B, TAIL)])
        pltpu.sync_copy(ones_v.at[pl.ds(0, TAIL)],
                        deg_sh.at[pl.ds(rowbase + 4 * B, TAIL)])

        @pl.loop(0, B)
        def _(i):
            ones_v[i, :] = ones16

        plsc.subcore_barrier()

        def do_chunk(base, o, par, nc, last):
            sd = sds[par]
            nxt = sds[1 - par]
            for k in range(CH):
                buf, sem = rows[k % 2], gsems[k % 2]
                other, osem = rows[1 - k % 2], gsems[1 - k % 2]
                pltpu.make_async_copy(x_hbm.at[sd.at[k, 0]], buf, sem).wait()
                if k + 1 < CH:
                    pltpu.async_copy(x_hbm.at[sd.at[k + 1, 0]], other, osem)
                elif not last:
                    pltpu.make_async_copy(sd_hbm.at[0], nxt, isem).wait()
                    pltpu.async_copy(x_hbm.at[nxt.at[0, 0]], other, osem)
                pltpu.sync_copy(buf, acc_sh.at[sd.at[k, 1]], add=True)
                pltpu.sync_copy(ones_v, deg_sh.at[sd.at[k, 1]], add=True)
                if k + 1 == CH and not last:
                    @pl.when(o + 2 < nc)
                    def _():
                        pltpu.async_copy(sd_hbm.at[base + o + 2], sd, isem)

        def pipeline(nc, base):
            pltpu.sync_copy(sd_hbm.at[base], sd0)
            if nc > 1:
                pltpu.async_copy(sd_hbm.at[base + 1], sd1, isem)
            pltpu.async_copy(x_hbm.at[sd0.at[0, 0]], rows0, gsem0)
            if nc == 1:
                do_chunk(base, 0, 0, nc, True)
            elif nc % 2 == 1:
                @pl.loop(0, nc - 1, step=2)
                def _(o):
                    do_chunk(base, o, 0, nc, False)
                    do_chunk(base, o + 1, 1, nc, False)

                do_chunk(base, nc - 1, 0, nc, True)
            else:
                @pl.loop(0, nc - 2, step=2)
                def _(o):
                    do_chunk(base, o, 0, nc, False)
                    do_chunk(base, o + 1, 1, nc, False)

                do_chunk(base, nc - 2, 0, nc, False)
                do_chunk(base, nc - 1, 1, nc, True)

        @pl.when(cid == 0)
        def _():
            pipeline(k0, sid * k0)

        @pl.when(cid == 1)
        def _():
            pipeline(k1, NUM_SUBCORES * k0 + sid * k1)

        plsc.subcore_barrier()

        for c in range(4):
            pltpu.sync_copy(acc_sh.at[pl.ds(rowbase + c * B, B)], rows0)
            pltpu.sync_copy(rows0, acc_hbm.at[cid, pl.ds(rowbase + c * B, B)])
            pltpu.sync_copy(deg_sh.at[pl.ds(rowbase + c * B, B)], ones_v)
            pltpu.sync_copy(ones_v, deg_hbm.at[cid, pl.ds(rowbase + c * B, B)])
        pltpu.sync_copy(acc_sh.at[pl.ds(rowbase + 4 * B, TAIL)],
                        rows0.at[pl.ds(0, TAIL)])
        pltpu.sync_copy(rows0.at[pl.ds(0, TAIL)],
                        acc_hbm.at[cid, pl.ds(rowbase + 4 * B, TAIL)])
        pltpu.sync_copy(deg_sh.at[pl.ds(rowbase + 4 * B, TAIL)],
                        ones_v.at[pl.ds(0, TAIL)])
        pltpu.sync_copy(ones_v.at[pl.ds(0, TAIL)],
                        deg_hbm.at[cid, pl.ds(rowbase + 4 * B, TAIL)])

    return sc_kernel


def _xr_body(x_ref, r_ref, b_ref, o_ref):
    o_ref[...] = (
        jnp.dot(x_ref[...], r_ref[...], preferred_element_type=jnp.float32)
        + b_ref[...]
    )


def _tc_body(acc_ref, deg_ref, xr_ref, w_ref, o_ref):
    acc = acc_ref[0, 0:N_NODES, :] + acc_ref[1, 0:N_NODES, :]
    dg = deg_ref[0] + deg_ref[1]
    grp = D // DEG_W
    sel = (lax.broadcasted_iota(jnp.int32, (D, grp), 0)
           == DEG_W * lax.broadcasted_iota(jnp.int32, (D, grp), 1))
    deg8 = jnp.dot(dg, sel.astype(jnp.float32),
                   preferred_element_type=jnp.float32)
    scale8 = 1.0 / jnp.maximum(deg8[0:N_NODES // grp], 1.0)
    scaled = (acc.reshape(N_NODES // grp, grp, D)
              * scale8[:, :, None]).reshape(N_NODES, D)
    o_ref[...] = (
        jnp.dot(scaled, w_ref[...], preferred_element_type=jnp.float32)
        + xr_ref[...]
    )


def kernel(x, edge_index, edge_attr, weight, root, bias):
    del edge_attr
    e = edge_index.shape[1]
    blk = CH * B
    npair = max(2, -(-e // (NUM_SUBCORES * blk)))
    k0 = min(npair - 1, max(1, round(npair * CORE0_FRAC)))
    k1 = npair - k0
    cap = NUM_SUBCORES * npair * blk
    pad = cap - e
    cyc = jnp.arange(pad, dtype=jnp.int32) % (N_PAD - N_NODES)
    nblk = cap // blk
    sd_real = edge_index.astype(jnp.int32).reshape(2, e // B, B)
    sd_pad = jnp.stack([cyc, N_NODES + cyc]).reshape(2, pad // B, B)
    sd = jnp.concatenate([sd_real.transpose(1, 0, 2),
                          sd_pad.transpose(1, 0, 2)])
    sd = sd.reshape(nblk, CH, 2, B)

    xr = pl.pallas_call(
        _xr_body,
        out_shape=jax.ShapeDtypeStruct((N_NODES, D), jnp.float32),
    )(x, root, bias.reshape(1, D))

    acc, deg = _sc_phase(k0, k1)(x, sd)

    deg_lin = deg.reshape(NUM_CORES, N_PAD * DEG_W // D, D)
    out = pl.pallas_call(
        _tc_body,
        out_shape=jax.ShapeDtypeStruct((N_NODES, D), jnp.float32),
    )(acc, deg_lin, xr, weight[0])
    return out
